# Initial kernel scaffold; baseline (speedup 1.0000x reference)
import jax
import jax.numpy as jnp
from jax.experimental import pallas as pl


def kernel(pos, atom_types, edge_index, W_embed, W_rad1, b_rad1, W_rad2, b_rad2, W_msg1, W_self1, W_msg2, W_self2, W_out1, W_out2, species_scale, species_shift):
    raise NotImplementedError("write your pallas kernel here")


# speedup vs baseline: 2.4282x; 2.4282x over previous
import functools

import jax
import jax.numpy as jnp
from jax import lax
from jax.experimental import pallas as pl
from jax.experimental.pallas import tpu as pltpu
from jax.experimental.pallas import tpu_sc as plsc

N = 10000
E = 320000
D = 128
NB = 8
NS = 10
H = 64
R_CUT = 5.0

NW = 32
E_TILE = E // NW
CHUNK = 80
NCHUNK = E_TILE // CHUNK
N_PAD = 10240
ROWS_PER_TILE = N_PAD // 16

_mesh = plsc.VectorSubcoreMesh(
    core_axis_name="c", subcore_axis_name="s", num_cores=2, num_subcores=16
)


_GEOM_KW = dict(
    out_type=jax.ShapeDtypeStruct((E,), jnp.float32),
    mesh=_mesh,
    compiler_params=pltpu.CompilerParams(needs_layout_passes=False),
    scratch_types=[
        pltpu.VMEM((3 * N,), jnp.float32),
        pltpu.VMEM((E_TILE,), jnp.int32),
        pltpu.VMEM((E_TILE,), jnp.int32),
        pltpu.VMEM((E_TILE,), jnp.float32),
    ],
)


def _sc_geom_body(pos_hbm, src_hbm, dst_hbm, r2_hbm, pos_v, src_v, dst_v, r2_v):
    cid = lax.axis_index("c")
    sid = lax.axis_index("s")
    wid = cid * 16 + sid
    base = wid * E_TILE
    pltpu.sync_copy(pos_hbm, pos_v)
    pltpu.sync_copy(src_hbm.at[pl.ds(base, E_TILE)], src_v)
    pltpu.sync_copy(dst_hbm.at[pl.ds(base, E_TILE)], dst_v)

    def body(j, carry):
        sl = pl.ds(j * 16, 16)
        s3 = src_v[sl] * 3
        d3 = dst_v[sl] * 3
        acc = jnp.zeros((16,), jnp.float32)
        for c in range(3):
            ps = plsc.load_gather(pos_v, [s3 + c])
            pd = plsc.load_gather(pos_v, [d3 + c])
            df = ps - pd
            acc = acc + df * df
        r2_v[sl] = acc
        return carry

    lax.fori_loop(0, E_TILE // 16, body, 0)
    pltpu.sync_copy(r2_v, r2_hbm.at[pl.ds(base, E_TILE)])


_sc_geom = pl.kernel(_sc_geom_body, **_GEOM_KW)


_AGG_KW = dict(
    out_type=(
        jax.ShapeDtypeStruct((N_PAD, D), jnp.float32),
        jax.ShapeDtypeStruct((N_PAD, D), jnp.float32),
    ),
    mesh=_mesh,
    compiler_params=pltpu.CompilerParams(needs_layout_passes=False),
    scratch_types=[
        pltpu.VMEM_SHARED((N_PAD, D), jnp.float32),
        pltpu.VMEM((CHUNK, D), jnp.float32),
        pltpu.VMEM((CHUNK, D), jnp.float32),
        pltpu.VMEM((CHUNK,), jnp.int32),
        pltpu.VMEM((CHUNK,), jnp.int32),
        pltpu.VMEM((80, D), jnp.float32),
    ],
)


def _sc_agg_body(xw_hbm, rad_hbm, src_hbm, dst_hbm, out0, out1,
                 acc, xw_v, rad_v, sidx_v, didx_v, zero_v):
    cid = lax.axis_index("c")
    sid = lax.axis_index("s")
    wid = cid * 16 + sid
    base = wid * E_TILE

    def zbody(i, carry):
        for c in range(D // 16):
            zero_v[i, pl.ds(c * 16, 16)] = jnp.zeros((16,), jnp.float32)
        return carry

    lax.fori_loop(0, 80, zbody, 0)
    for k in range(ROWS_PER_TILE // 80):
        pltpu.sync_copy(zero_v, acc.at[pl.ds(sid * ROWS_PER_TILE + k * 80, 80)])
    plsc.subcore_barrier()

    def chunk_body(j, carry):
        eb = base + j * CHUNK
        pltpu.sync_copy(src_hbm.at[pl.ds(eb, CHUNK)], sidx_v)
        pltpu.sync_copy(dst_hbm.at[pl.ds(eb, CHUNK)], didx_v)
        pltpu.sync_copy(xw_hbm.at[sidx_v], xw_v)
        pltpu.sync_copy(rad_hbm.at[pl.ds(eb, CHUNK)], rad_v)

        def mbody(r, c2):
            for c in range(D // 16):
                sl = pl.ds(c * 16, 16)
                xw_v[r, sl] = xw_v[r, sl] * rad_v[r, sl]
            return c2

        lax.fori_loop(0, CHUNK, mbody, 0)
        pltpu.sync_copy(xw_v, acc.at[didx_v], add=True)
        return carry

    lax.fori_loop(0, NCHUNK, chunk_body, 0)
    plsc.subcore_barrier()

    rows = pl.ds(sid * ROWS_PER_TILE, ROWS_PER_TILE)

    @pl.when(cid == 0)
    def _():
        pltpu.sync_copy(acc.at[rows], out0.at[rows])

    @pl.when(cid == 1)
    def _():
        pltpu.sync_copy(acc.at[rows], out1.at[rows])


_sc_agg = pl.kernel(_sc_agg_body, **_AGG_KW)


def _silu(v):
    return v * (1.0 / (1.0 + jnp.exp(-v)))


CE = 4000


def _radial_body(r2_ref, w1_ref, b1_ref, w2_ref, b2_ref, out_ref):
    r2 = r2_ref[...]
    r = jnp.sqrt(r2 + 1e-12)
    nvec = lax.broadcasted_iota(jnp.int32, (1, NB), 1).astype(jnp.float32) + 1.0
    basis = jnp.sqrt(2.0 / R_CUT) * jnp.sin(nvec * (jnp.pi / R_CUT) * r) / r
    u = r / R_CUT
    u2 = u * u
    u3 = u2 * u
    u6 = u3 * u3
    u7 = u6 * u
    u8 = u7 * u
    env = 1.0 - 28.0 * u6 + 48.0 * u7 - 21.0 * u8
    env = jnp.where(u < 1.0, env, 0.0)
    basis = basis * env
    hm = jnp.dot(basis, w1_ref[...], preferred_element_type=jnp.float32)
    hm = _silu(hm + b1_ref[...])
    out_ref[...] = (
        jnp.dot(hm, w2_ref[...], preferred_element_type=jnp.float32) + b2_ref[...]
    )


_RADIAL_KW = dict(
    grid=(E // CE,),
    in_specs=[
        pl.BlockSpec((CE, 1), lambda i: (i, 0)),
        pl.BlockSpec((NB, H), lambda i: (0, 0)),
        pl.BlockSpec((1, H), lambda i: (0, 0)),
        pl.BlockSpec((H, D), lambda i: (0, 0)),
        pl.BlockSpec((1, D), lambda i: (0, 0)),
    ],
    out_specs=pl.BlockSpec((CE, D), lambda i: (i, 0)),
    out_shape=jax.ShapeDtypeStruct((E, D), jnp.float32),
)
_tc_radial = pl.pallas_call(_radial_body, **_RADIAL_KW)

CN = 1000


def _embed_body(t_ref, we_ref, wm_ref, x_ref, xw_ref):
    t = t_ref[...]
    io = lax.broadcasted_iota(jnp.int32, (CN, 16), 1)
    oh = jnp.where(io == t, 1.0, 0.0)
    x = jnp.dot(oh, we_ref[...], preferred_element_type=jnp.float32)
    x_ref[...] = x
    xw_ref[...] = jnp.dot(x, wm_ref[...], preferred_element_type=jnp.float32)


_EMBED_KW = dict(
    grid=(N // CN,),
    in_specs=[
        pl.BlockSpec((CN, 1), lambda i: (i, 0)),
        pl.BlockSpec((16, D), lambda i: (0, 0)),
        pl.BlockSpec((D, D), lambda i: (0, 0)),
    ],
    out_specs=(
        pl.BlockSpec((CN, D), lambda i: (i, 0)),
        pl.BlockSpec((CN, D), lambda i: (i, 0)),
    ),
    out_shape=(
        jax.ShapeDtypeStruct((N, D), jnp.float32),
        jax.ShapeDtypeStruct((N, D), jnp.float32),
    ),
)
_tc_embed = pl.pallas_call(_embed_body, **_EMBED_KW)


def _update_body(x_ref, a0_ref, a1_ref, ws_ref, wm_ref, x1_ref, xw2_ref):
    x = x_ref[...]
    pre = (
        jnp.dot(x, ws_ref[...], preferred_element_type=jnp.float32)
        + a0_ref[...] + a1_ref[...]
    )
    x1 = _silu(pre)
    x1_ref[...] = x1
    xw2_ref[...] = jnp.dot(x1, wm_ref[...], preferred_element_type=jnp.float32)


_UPDATE_KW = dict(
    grid=(N // CN,),
    in_specs=[
        pl.BlockSpec((CN, D), lambda i: (i, 0)),
        pl.BlockSpec((CN, D), lambda i: (i, 0)),
        pl.BlockSpec((CN, D), lambda i: (i, 0)),
        pl.BlockSpec((D, D), lambda i: (0, 0)),
        pl.BlockSpec((D, D), lambda i: (0, 0)),
    ],
    out_specs=(
        pl.BlockSpec((CN, D), lambda i: (i, 0)),
        pl.BlockSpec((CN, D), lambda i: (i, 0)),
    ),
    out_shape=(
        jax.ShapeDtypeStruct((N, D), jnp.float32),
        jax.ShapeDtypeStruct((N, D), jnp.float32),
    ),
)
_tc_update = pl.pallas_call(_update_body, **_UPDATE_KW)


def _final_body(x_ref, a0_ref, a1_ref, t_ref, ws_ref, wo1_ref, wo2_ref,
                sc_ref, sh_ref, out_ref):
    i = pl.program_id(0)
    x = x_ref[...]
    pre = (
        jnp.dot(x, ws_ref[...], preferred_element_type=jnp.float32)
        + a0_ref[...] + a1_ref[...]
    )
    x2 = _silu(pre)
    hm = _silu(jnp.dot(x2, wo1_ref[...], preferred_element_type=jnp.float32))
    e = jnp.dot(hm, wo2_ref[...], preferred_element_type=jnp.float32)
    t = t_ref[...]
    io = lax.broadcasted_iota(jnp.int32, (CN, 16), 1)
    oh = jnp.where(io == t, 1.0, 0.0)
    scale = jnp.dot(oh, sc_ref[...], preferred_element_type=jnp.float32)
    shift = jnp.dot(oh, sh_ref[...], preferred_element_type=jnp.float32)
    e = e * scale + shift
    s = jnp.sum(e, axis=0, keepdims=True)

    @pl.when(i == 0)
    def _():
        out_ref[...] = jnp.zeros((1, 1), jnp.float32)

    out_ref[...] += s


_FINAL_KW = dict(
    grid=(N // CN,),
    in_specs=[
        pl.BlockSpec((CN, D), lambda i: (i, 0)),
        pl.BlockSpec((CN, D), lambda i: (i, 0)),
        pl.BlockSpec((CN, D), lambda i: (i, 0)),
        pl.BlockSpec((CN, 1), lambda i: (i, 0)),
        pl.BlockSpec((D, D), lambda i: (0, 0)),
        pl.BlockSpec((D, H), lambda i: (0, 0)),
        pl.BlockSpec((H, 1), lambda i: (0, 0)),
        pl.BlockSpec((16, 1), lambda i: (0, 0)),
        pl.BlockSpec((16, 1), lambda i: (0, 0)),
    ],
    out_specs=pl.BlockSpec((1, 1), lambda i: (0, 0)),
    out_shape=jax.ShapeDtypeStruct((1, 1), jnp.float32),
)
_tc_final = pl.pallas_call(_final_body, **_FINAL_KW)


def kernel(pos, atom_types, edge_index, W_embed, W_rad1, b_rad1, W_rad2, b_rad2,
           W_msg1, W_self1, W_msg2, W_self2, W_out1, W_out2,
           species_scale, species_shift):
    src = edge_index[0].astype(jnp.int32)
    dst = edge_index[1].astype(jnp.int32)
    pos_flat = pos.reshape(-1).astype(jnp.float32)

    r2 = _sc_geom(pos_flat, src, dst)
    radial = _tc_radial(r2.reshape(E, 1), W_rad1, b_rad1.reshape(1, H),
                        W_rad2, b_rad2.reshape(1, D))

    t2 = atom_types.astype(jnp.int32).reshape(N, 1)
    we_pad = jnp.zeros((16, D), jnp.float32).at[:NS].set(W_embed)
    x0, xw1 = _tc_embed(t2, we_pad, W_msg1)

    a0, a1 = _sc_agg(xw1, radial, src, dst)
    x1, xw2 = _tc_update(x0, a0, a1, W_self1, W_msg2)

    b0, b1 = _sc_agg(xw2, radial, src, dst)

    sc_pad = jnp.zeros((16, 1), jnp.float32).at[:NS, 0].set(species_scale)
    sh_pad = jnp.zeros((16, 1), jnp.float32).at[:NS, 0].set(species_shift)
    tot = _tc_final(x1, b0, b1, t2, W_self2, W_out1, W_out2, sc_pad, sh_pad)
    return tot.reshape(())
